# baseline (speedup 1.0000x reference)
import functools

import jax
import jax.numpy as jnp
from jax import lax
from jax.experimental import pallas as pl
from jax.experimental.pallas import tpu as pltpu
from jax.experimental.pallas import tpu_sc as plsc

N_COARSE = 4096
N_FINE = 16384
NIN = 512
NSKIP = 256
NOUT = 512
HID = (NIN + NSKIP + NOUT) // 2
KNN = 3

BF = 512
BM = 512

_NC = 2
_NS = 16
_NW = _NC * _NS
_PW = N_FINE // _NW
_CH = 16
_NCHUNK = _PW // _CH


def _knn_body(ps_ref, posT_ref, idx_ref, w_ref):
    ps = ps_ref[...]
    posT = posT_ref[...]
    dot = lax.dot_general(ps, posT, (((1,), (0,)), ((), ())),
                          preferred_element_type=jnp.float32)
    p2 = jnp.sum(posT * posT, axis=0, keepdims=True)
    ps2 = jnp.sum(ps * ps, axis=1, keepdims=True)
    d2 = jnp.maximum(ps2 + p2 - 2.0 * dot, 0.0)
    big = jnp.float32(3.0e38)
    nlane = 128
    nchk = N_COARSE // nlane
    av = jnp.full((d2.shape[0], nlane), big, jnp.float32)
    bv = av
    cv = av
    ai = jnp.zeros((d2.shape[0], nlane), jnp.int32)
    bi = ai
    ci = ai
    for c in range(nchk):
        xv = lax.slice(d2, (0, c * nlane), (d2.shape[0], (c + 1) * nlane))
        xc = jnp.int32(c)
        c1 = xv < av
        tv = jnp.where(c1, av, xv)
        ti = jnp.where(c1, ai, xc)
        av = jnp.minimum(av, xv)
        ai = jnp.where(c1, xc, ai)
        c2 = tv < bv
        uv = jnp.where(c2, bv, tv)
        ui = jnp.where(c2, bi, ti)
        bv = jnp.minimum(bv, tv)
        bi = jnp.where(c2, ti, bi)
        c3 = uv < cv
        cv = jnp.minimum(cv, uv)
        ci = jnp.where(c3, ui, ci)
    vals3 = jnp.concatenate([av, bv, cv], axis=1)
    lane = jnp.bitwise_and(
        lax.broadcasted_iota(jnp.int32, vals3.shape, 1), nlane - 1)
    gidx = jnp.concatenate([ai, bi, ci], axis=1) * nlane + lane
    inds, vals = [], []
    for _ in range(KNN):
        m = jnp.min(vals3, axis=1, keepdims=True)
        cand = jnp.where(vals3 == m, gidx, jnp.int32(2**30))
        g = jnp.min(cand, axis=1, keepdims=True)
        inds.append(g)
        vals.append(m)
        vals3 = jnp.where(gidx == g, big, vals3)
    ws = [1.0 / jnp.maximum(v, 1e-16) for v in vals]
    wsum = ws[0] + ws[1] + ws[2]
    idx_ref[...] = jnp.concatenate(inds, axis=1)
    w_ref[...] = jnp.concatenate([w / wsum for w in ws], axis=1)


def _knn(pos_skip, posT):
    n = pos_skip.shape[0]
    return pl.pallas_call(
        _knn_body,
        grid=(n // BF,),
        in_specs=[
            pl.BlockSpec((BF, 3), lambda i: (i, 0)),
            pl.BlockSpec((3, N_COARSE), lambda i: (0, 0)),
        ],
        out_specs=[
            pl.BlockSpec((BF, KNN), lambda i: (i, 0)),
            pl.BlockSpec((BF, KNN), lambda i: (i, 0)),
        ],
        out_shape=[
            jax.ShapeDtypeStruct((n, KNN), jnp.int32),
            jax.ShapeDtypeStruct((n, KNN), jnp.float32),
        ],
    )(pos_skip, posT)


_NBUF = 4


def _gather_body(pw, x_hbm, idx_hbm, w_hbm, xi_hbm, idxv, wv,
                 rows0, rows1, rows2, rows3, outb0, outb1,
                 gsem0, gsem1, gsem2, gsem3, ssem0, ssem1):
    nchunk = pw // _CH
    wid = lax.axis_index("s") * _NC + lax.axis_index("c")
    base = wid * pw
    pltpu.sync_copy(idx_hbm.at[pl.ds(base * KNN, KNN * pw)], idxv)
    pltpu.sync_copy(w_hbm.at[pl.ds(base * KNN, KNN * pw)],
                    wv.at[pl.ds(0, KNN * pw)])

    rows = (rows0, rows1, rows2, rows3)
    outb = (outb0, outb1)
    gsem = (gsem0, gsem1, gsem2, gsem3)
    ssem = (ssem0, ssem1)

    def issue_gather(ch, b):
        pltpu.async_copy(
            x_hbm.at[idxv.at[pl.ds(KNN * _CH * ch, KNN * _CH)]],
            rows[b], gsem[b])

    def gather_wait(b):
        pltpu.make_async_copy(x_hbm.at[pl.ds(0, KNN * _CH)], rows[b],
                              gsem[b]).wait()

    def store_wait(ob):
        pltpu.make_async_copy(outb[ob], xi_hbm.at[pl.ds(0, _CH)],
                              ssem[ob]).wait()

    for b in range(_NBUF):
        issue_gather(b, b)

    def quad_body(p, carry):
        for b in range(_NBUF):
            ch = _NBUF * p + b
            cb = ch * _CH
            ob = b % 2
            gather_wait(b)

            @pl.when(ch >= 2)
            def _():
                store_wait(ob)

            def point_body(i, c2):
                i3 = KNN * i
                g = KNN * cb + i3
                v = wv[pl.ds(g, 16)]
                w0 = v[0]
                w1 = v[1]
                w2 = v[2]
                rb = rows[b]
                obr = outb[ob]
                for d in range(NIN // 16):
                    sl = pl.ds(d * 16, 16)
                    obr[i, sl] = (rb[i3, sl] * w0 + rb[i3 + 1, sl] * w1
                                  + rb[i3 + 2, sl] * w2)
                return c2

            lax.fori_loop(0, _CH, point_body, 0)

            @pl.when(ch + _NBUF < nchunk)
            def _():
                issue_gather(ch + _NBUF, b)

            pltpu.async_copy(outb[ob], xi_hbm.at[pl.ds(base + cb, _CH)],
                             ssem[ob])
        return carry

    lax.fori_loop(0, nchunk // _NBUF, quad_body, 0)
    store_wait(0)
    store_wait(1)


@functools.lru_cache(maxsize=None)
def _gather_combine_fn(npoints):
    pw = npoints // _NW
    return functools.partial(
        pl.kernel,
        mesh=plsc.VectorSubcoreMesh(core_axis_name="c", subcore_axis_name="s"),
        out_type=jax.ShapeDtypeStruct((npoints, NIN), jnp.float32),
        scratch_types=[
            pltpu.VMEM((KNN * pw,), jnp.int32),
            pltpu.VMEM((KNN * pw + 16,), jnp.float32),
            pltpu.VMEM((KNN * _CH, NIN), jnp.float32),
            pltpu.VMEM((KNN * _CH, NIN), jnp.float32),
            pltpu.VMEM((KNN * _CH, NIN), jnp.float32),
            pltpu.VMEM((KNN * _CH, NIN), jnp.float32),
            pltpu.VMEM((_CH, NIN), jnp.float32),
            pltpu.VMEM((_CH, NIN), jnp.float32),
            pltpu.SemaphoreType.DMA,
            pltpu.SemaphoreType.DMA,
            pltpu.SemaphoreType.DMA,
            pltpu.SemaphoreType.DMA,
            pltpu.SemaphoreType.DMA,
            pltpu.SemaphoreType.DMA,
        ],
    )(functools.partial(_gather_body, pw))


def _mlp1_body(xi_ref, xs_ref, w1a_ref, w1b_ref, b1_ref, h1_ref, s_ref, q_ref):
    h = jnp.dot(xi_ref[...], w1a_ref[...], preferred_element_type=jnp.float32)
    h = h + jnp.dot(xs_ref[...], w1b_ref[...],
                    preferred_element_type=jnp.float32)
    h = jnp.maximum(h + b1_ref[...], 0.0)
    h1_ref[...] = h

    @pl.when(pl.program_id(0) == 0)
    def _():
        s_ref[...] = jnp.zeros_like(s_ref)
        q_ref[...] = jnp.zeros_like(q_ref)

    s_ref[...] += jnp.sum(h, axis=0, keepdims=True)
    q_ref[...] += jnp.sum(h * h, axis=0, keepdims=True)


def _mlp1(xi, x_skip, w1a, w1b, b1):
    n = xi.shape[0]
    return pl.pallas_call(
        _mlp1_body,
        grid=(n // BM,),
        in_specs=[
            pl.BlockSpec((BM, NIN), lambda i: (i, 0)),
            pl.BlockSpec((BM, NSKIP), lambda i: (i, 0)),
            pl.BlockSpec((NIN, HID), lambda i: (0, 0)),
            pl.BlockSpec((NSKIP, HID), lambda i: (0, 0)),
            pl.BlockSpec((1, HID), lambda i: (0, 0)),
        ],
        out_specs=[
            pl.BlockSpec((BM, HID), lambda i: (i, 0)),
            pl.BlockSpec((1, HID), lambda i: (0, 0)),
            pl.BlockSpec((1, HID), lambda i: (0, 0)),
        ],
        out_shape=[
            jax.ShapeDtypeStruct((n, HID), jnp.float32),
            jax.ShapeDtypeStruct((1, HID), jnp.float32),
            jax.ShapeDtypeStruct((1, HID), jnp.float32),
        ],
    )(xi, x_skip, w1a, w1b, b1)


def _mlp2_body(h1_ref, s_ref, q_ref, g1_ref, be1_ref, w2_ref, b2_ref,
               h2_ref, s2_ref, q2_ref):
    n = jnp.float32(N_FINE)
    mean = s_ref[...] / n
    var = q_ref[...] / n - mean * mean
    a = g1_ref[...] * lax.rsqrt(var + 1e-5)
    c = be1_ref[...] - a * mean
    hn = h1_ref[...] * a + c
    h = jnp.dot(hn, w2_ref[...], preferred_element_type=jnp.float32)
    h = jnp.maximum(h + b2_ref[...], 0.0)
    h2_ref[...] = h

    @pl.when(pl.program_id(0) == 0)
    def _():
        s2_ref[...] = jnp.zeros_like(s2_ref)
        q2_ref[...] = jnp.zeros_like(q2_ref)

    s2_ref[...] += jnp.sum(h, axis=0, keepdims=True)
    q2_ref[...] += jnp.sum(h * h, axis=0, keepdims=True)


def _mlp2(h1, s1, q1, g1, be1, w2, b2):
    return pl.pallas_call(
        _mlp2_body,
        grid=(N_FINE // BM,),
        in_specs=[
            pl.BlockSpec((BM, HID), lambda i: (i, 0)),
            pl.BlockSpec((1, HID), lambda i: (0, 0)),
            pl.BlockSpec((1, HID), lambda i: (0, 0)),
            pl.BlockSpec((1, HID), lambda i: (0, 0)),
            pl.BlockSpec((1, HID), lambda i: (0, 0)),
            pl.BlockSpec((HID, NOUT), lambda i: (0, 0)),
            pl.BlockSpec((1, NOUT), lambda i: (0, 0)),
        ],
        out_specs=[
            pl.BlockSpec((BM, NOUT), lambda i: (i, 0)),
            pl.BlockSpec((1, NOUT), lambda i: (0, 0)),
            pl.BlockSpec((1, NOUT), lambda i: (0, 0)),
        ],
        out_shape=[
            jax.ShapeDtypeStruct((N_FINE, NOUT), jnp.float32),
            jax.ShapeDtypeStruct((1, NOUT), jnp.float32),
            jax.ShapeDtypeStruct((1, NOUT), jnp.float32),
        ],
    )(h1, s1, q1, g1, be1, w2, b2)


def _bn2_body(h2_ref, s_ref, q_ref, g2_ref, be2_ref, o_ref):
    n = jnp.float32(N_FINE)
    mean = s_ref[...] / n
    var = q_ref[...] / n - mean * mean
    a = g2_ref[...] * lax.rsqrt(var + 1e-5)
    c = be2_ref[...] - a * mean
    o_ref[...] = h2_ref[...] * a + c


def _bn2(h2, s2, q2, g2, be2):
    return pl.pallas_call(
        _bn2_body,
        grid=(N_FINE // BM,),
        in_specs=[
            pl.BlockSpec((BM, NOUT), lambda i: (i, 0)),
            pl.BlockSpec((1, NOUT), lambda i: (0, 0)),
            pl.BlockSpec((1, NOUT), lambda i: (0, 0)),
            pl.BlockSpec((1, NOUT), lambda i: (0, 0)),
            pl.BlockSpec((1, NOUT), lambda i: (0, 0)),
        ],
        out_specs=pl.BlockSpec((BM, NOUT), lambda i: (i, 0)),
        out_shape=jax.ShapeDtypeStruct((N_FINE, NOUT), jnp.float32),
    )(h2, s2, q2, g2, be2)


def kernel(x, pos, batch, x_skip, pos_skip, batch_skip,
           W1, b1, g1, be1, W2, b2, g2, be2):
    posT = pos.T
    bounds = [0, N_FINE // 2, N_FINE]
    w1a, w1b = W1[:NIN], W1[NIN:]
    b1r = b1.reshape(1, HID)
    xis = []
    for lo, hi in zip(bounds[:-1], bounds[1:]):
        sl = slice(lo, hi)
        idx, w = _knn(pos_skip[sl], posT)
        xis.append((_gather_combine_fn(hi - lo)(x, idx.reshape(-1),
                                                w.reshape(-1)), sl))
    parts = [_mlp1(xi, x_skip[sl], w1a, w1b, b1r) for xi, sl in xis]
    h1 = jnp.concatenate([p[0] for p in parts], axis=0)
    s1 = sum(p[1] for p in parts)
    q1 = sum(p[2] for p in parts)
    h2, s2, q2 = _mlp2(h1, s1, q1, g1.reshape(1, HID), be1.reshape(1, HID),
                       W2, b2.reshape(1, NOUT))
    out = _bn2(h2, s2, q2, g2.reshape(1, NOUT), be2.reshape(1, NOUT))
    return (out, pos_skip, batch_skip)


# speedup vs baseline: 1.0127x; 1.0127x over previous
import functools

import jax
import jax.numpy as jnp
from jax import lax
from jax.experimental import pallas as pl
from jax.experimental.pallas import tpu as pltpu
from jax.experimental.pallas import tpu_sc as plsc

N_COARSE = 4096
N_FINE = 16384
NIN = 512
NSKIP = 256
NOUT = 512
HID = (NIN + NSKIP + NOUT) // 2
KNN = 3

BF = 512
BM = 512

_NC = 2
_NS = 16
_NW = _NC * _NS
_PW = N_FINE // _NW
_CH = 16
_NCHUNK = _PW // _CH


def _knn_body(ps_ref, posT_ref, idx_ref, w_ref):
    ps = ps_ref[...]
    posT = posT_ref[...]
    dot = lax.dot_general(ps, posT, (((1,), (0,)), ((), ())),
                          preferred_element_type=jnp.float32)
    p2 = jnp.sum(posT * posT, axis=0, keepdims=True)
    ps2 = jnp.sum(ps * ps, axis=1, keepdims=True)
    d2 = jnp.maximum(ps2 + p2 - 2.0 * dot, 0.0)
    big = jnp.float32(3.0e38)
    nlane = 128
    nchk = N_COARSE // nlane
    av = jnp.full((d2.shape[0], nlane), big, jnp.float32)
    bv = av
    cv = av
    ai = jnp.zeros((d2.shape[0], nlane), jnp.int32)
    bi = ai
    ci = ai
    for c in range(nchk):
        xv = lax.slice(d2, (0, c * nlane), (d2.shape[0], (c + 1) * nlane))
        xc = jnp.int32(c)
        c1 = xv < av
        tv = jnp.where(c1, av, xv)
        ti = jnp.where(c1, ai, xc)
        av = jnp.minimum(av, xv)
        ai = jnp.where(c1, xc, ai)
        c2 = tv < bv
        uv = jnp.where(c2, bv, tv)
        ui = jnp.where(c2, bi, ti)
        bv = jnp.minimum(bv, tv)
        bi = jnp.where(c2, ti, bi)
        c3 = uv < cv
        cv = jnp.minimum(cv, uv)
        ci = jnp.where(c3, ui, ci)
    vals3 = jnp.concatenate([av, bv, cv], axis=1)
    lane = jnp.bitwise_and(
        lax.broadcasted_iota(jnp.int32, vals3.shape, 1), nlane - 1)
    gidx = jnp.concatenate([ai, bi, ci], axis=1) * nlane + lane
    inds, vals = [], []
    for _ in range(KNN):
        m = jnp.min(vals3, axis=1, keepdims=True)
        cand = jnp.where(vals3 == m, gidx, jnp.int32(2**30))
        g = jnp.min(cand, axis=1, keepdims=True)
        inds.append(g)
        vals.append(m)
        vals3 = jnp.where(gidx == g, big, vals3)
    ws = [1.0 / jnp.maximum(v, 1e-16) for v in vals]
    wsum = ws[0] + ws[1] + ws[2]
    idx_ref[...] = jnp.concatenate(inds, axis=1)
    w_ref[...] = jnp.concatenate([w / wsum for w in ws], axis=1)


def _knn(pos_skip, posT):
    n = pos_skip.shape[0]
    return pl.pallas_call(
        _knn_body,
        grid=(n // BF,),
        in_specs=[
            pl.BlockSpec((BF, 3), lambda i: (i, 0)),
            pl.BlockSpec((3, N_COARSE), lambda i: (0, 0)),
        ],
        out_specs=[
            pl.BlockSpec((BF, KNN), lambda i: (i, 0)),
            pl.BlockSpec((BF, KNN), lambda i: (i, 0)),
        ],
        out_shape=[
            jax.ShapeDtypeStruct((n, KNN), jnp.int32),
            jax.ShapeDtypeStruct((n, KNN), jnp.float32),
        ],
    )(pos_skip, posT)


_NBUF = 2


def _gather_body(pw, x_hbm, idx_hbm, w_hbm, xi_hbm, idxv, wv, *scr):
    nchunk = pw // _CH
    wid = lax.axis_index("s") * _NC + lax.axis_index("c")
    base = wid * pw
    pltpu.sync_copy(idx_hbm.at[pl.ds(base * KNN, KNN * pw)], idxv)
    pltpu.sync_copy(w_hbm.at[pl.ds(base * KNN, KNN * pw)],
                    wv.at[pl.ds(0, KNN * pw)])

    rows = scr[:_NBUF]
    outb = scr[_NBUF:_NBUF + 2]
    gsem = scr[_NBUF + 2:2 * _NBUF + 2]
    ssem = scr[2 * _NBUF + 2:]

    def issue_gather(ch, b):
        pltpu.async_copy(
            x_hbm.at[idxv.at[pl.ds(KNN * _CH * ch, KNN * _CH)]],
            rows[b], gsem[b])

    def gather_wait(b):
        pltpu.make_async_copy(x_hbm.at[pl.ds(0, KNN * _CH)], rows[b],
                              gsem[b]).wait()

    def store_wait(ob):
        pltpu.make_async_copy(outb[ob], xi_hbm.at[pl.ds(0, _CH)],
                              ssem[ob]).wait()

    for b in range(_NBUF):
        issue_gather(b, b)

    def quad_body(p, carry):
        for b in range(_NBUF):
            ch = _NBUF * p + b
            cb = ch * _CH
            ob = b % 2
            gather_wait(b)

            @pl.when(ch >= 2)
            def _():
                store_wait(ob)

            def point_body(i, c2):
                i3 = KNN * i
                g = KNN * cb + i3
                v = wv[pl.ds(g, 16)]
                w0 = v[0]
                w1 = v[1]
                w2 = v[2]
                rb = rows[b]
                obr = outb[ob]
                for d in range(NIN // 16):
                    sl = pl.ds(d * 16, 16)
                    obr[i, sl] = (rb[i3, sl] * w0 + rb[i3 + 1, sl] * w1
                                  + rb[i3 + 2, sl] * w2)
                return c2

            lax.fori_loop(0, _CH, point_body, 0)

            @pl.when(ch + _NBUF < nchunk)
            def _():
                issue_gather(ch + _NBUF, b)

            pltpu.async_copy(outb[ob], xi_hbm.at[pl.ds(base + cb, _CH)],
                             ssem[ob])
        return carry

    lax.fori_loop(0, nchunk // _NBUF, quad_body, 0)
    store_wait(0)
    store_wait(1)


@functools.lru_cache(maxsize=None)
def _gather_combine_fn(npoints):
    pw = npoints // _NW
    return functools.partial(
        pl.kernel,
        mesh=plsc.VectorSubcoreMesh(core_axis_name="c", subcore_axis_name="s"),
        out_type=jax.ShapeDtypeStruct((npoints, NIN), jnp.float32),
        scratch_types=[
            pltpu.VMEM((KNN * pw,), jnp.int32),
            pltpu.VMEM((KNN * pw + 16,), jnp.float32),
        ] + [pltpu.VMEM((KNN * _CH, NIN), jnp.float32)] * _NBUF + [
            pltpu.VMEM((_CH, NIN), jnp.float32),
            pltpu.VMEM((_CH, NIN), jnp.float32),
        ] + [pltpu.SemaphoreType.DMA] * (_NBUF + 2),
    )(functools.partial(_gather_body, pw))


def _mlp1_body(xi_ref, xs_ref, w1a_ref, w1b_ref, b1_ref, h1_ref, s_ref, q_ref):
    h = jnp.dot(xi_ref[...], w1a_ref[...], preferred_element_type=jnp.float32)
    h = h + jnp.dot(xs_ref[...], w1b_ref[...],
                    preferred_element_type=jnp.float32)
    h = jnp.maximum(h + b1_ref[...], 0.0)
    h1_ref[...] = h

    @pl.when(pl.program_id(0) == 0)
    def _():
        s_ref[...] = jnp.zeros_like(s_ref)
        q_ref[...] = jnp.zeros_like(q_ref)

    s_ref[...] += jnp.sum(h, axis=0, keepdims=True)
    q_ref[...] += jnp.sum(h * h, axis=0, keepdims=True)


def _mlp1(xi, x_skip, w1a, w1b, b1):
    n = xi.shape[0]
    return pl.pallas_call(
        _mlp1_body,
        grid=(n // BM,),
        in_specs=[
            pl.BlockSpec((BM, NIN), lambda i: (i, 0)),
            pl.BlockSpec((BM, NSKIP), lambda i: (i, 0)),
            pl.BlockSpec((NIN, HID), lambda i: (0, 0)),
            pl.BlockSpec((NSKIP, HID), lambda i: (0, 0)),
            pl.BlockSpec((1, HID), lambda i: (0, 0)),
        ],
        out_specs=[
            pl.BlockSpec((BM, HID), lambda i: (i, 0)),
            pl.BlockSpec((1, HID), lambda i: (0, 0)),
            pl.BlockSpec((1, HID), lambda i: (0, 0)),
        ],
        out_shape=[
            jax.ShapeDtypeStruct((n, HID), jnp.float32),
            jax.ShapeDtypeStruct((1, HID), jnp.float32),
            jax.ShapeDtypeStruct((1, HID), jnp.float32),
        ],
    )(xi, x_skip, w1a, w1b, b1)


def _mlp2_body(h1_ref, s_ref, q_ref, g1_ref, be1_ref, w2_ref, b2_ref,
               h2_ref, s2_ref, q2_ref):
    n = jnp.float32(N_FINE)
    mean = s_ref[...] / n
    var = q_ref[...] / n - mean * mean
    a = g1_ref[...] * lax.rsqrt(var + 1e-5)
    c = be1_ref[...] - a * mean
    hn = h1_ref[...] * a + c
    h = jnp.dot(hn, w2_ref[...], preferred_element_type=jnp.float32)
    h = jnp.maximum(h + b2_ref[...], 0.0)
    h2_ref[...] = h

    @pl.when(pl.program_id(0) == 0)
    def _():
        s2_ref[...] = jnp.zeros_like(s2_ref)
        q2_ref[...] = jnp.zeros_like(q2_ref)

    s2_ref[...] += jnp.sum(h, axis=0, keepdims=True)
    q2_ref[...] += jnp.sum(h * h, axis=0, keepdims=True)


def _mlp2(h1, s1, q1, g1, be1, w2, b2):
    return pl.pallas_call(
        _mlp2_body,
        grid=(N_FINE // BM,),
        in_specs=[
            pl.BlockSpec((BM, HID), lambda i: (i, 0)),
            pl.BlockSpec((1, HID), lambda i: (0, 0)),
            pl.BlockSpec((1, HID), lambda i: (0, 0)),
            pl.BlockSpec((1, HID), lambda i: (0, 0)),
            pl.BlockSpec((1, HID), lambda i: (0, 0)),
            pl.BlockSpec((HID, NOUT), lambda i: (0, 0)),
            pl.BlockSpec((1, NOUT), lambda i: (0, 0)),
        ],
        out_specs=[
            pl.BlockSpec((BM, NOUT), lambda i: (i, 0)),
            pl.BlockSpec((1, NOUT), lambda i: (0, 0)),
            pl.BlockSpec((1, NOUT), lambda i: (0, 0)),
        ],
        out_shape=[
            jax.ShapeDtypeStruct((N_FINE, NOUT), jnp.float32),
            jax.ShapeDtypeStruct((1, NOUT), jnp.float32),
            jax.ShapeDtypeStruct((1, NOUT), jnp.float32),
        ],
    )(h1, s1, q1, g1, be1, w2, b2)


def _bn2_body(h2_ref, s_ref, q_ref, g2_ref, be2_ref, o_ref):
    n = jnp.float32(N_FINE)
    mean = s_ref[...] / n
    var = q_ref[...] / n - mean * mean
    a = g2_ref[...] * lax.rsqrt(var + 1e-5)
    c = be2_ref[...] - a * mean
    o_ref[...] = h2_ref[...] * a + c


def _bn2(h2, s2, q2, g2, be2):
    return pl.pallas_call(
        _bn2_body,
        grid=(N_FINE // BM,),
        in_specs=[
            pl.BlockSpec((BM, NOUT), lambda i: (i, 0)),
            pl.BlockSpec((1, NOUT), lambda i: (0, 0)),
            pl.BlockSpec((1, NOUT), lambda i: (0, 0)),
            pl.BlockSpec((1, NOUT), lambda i: (0, 0)),
            pl.BlockSpec((1, NOUT), lambda i: (0, 0)),
        ],
        out_specs=pl.BlockSpec((BM, NOUT), lambda i: (i, 0)),
        out_shape=jax.ShapeDtypeStruct((N_FINE, NOUT), jnp.float32),
    )(h2, s2, q2, g2, be2)


def kernel(x, pos, batch, x_skip, pos_skip, batch_skip,
           W1, b1, g1, be1, W2, b2, g2, be2):
    posT = pos.T
    bounds = [0, N_FINE // 2, N_FINE]
    w1a, w1b = W1[:NIN], W1[NIN:]
    b1r = b1.reshape(1, HID)
    xis = []
    for lo, hi in zip(bounds[:-1], bounds[1:]):
        sl = slice(lo, hi)
        idx, w = _knn(pos_skip[sl], posT)
        xis.append((_gather_combine_fn(hi - lo)(x, idx.reshape(-1),
                                                w.reshape(-1)), sl))
    parts = [_mlp1(xi, x_skip[sl], w1a, w1b, b1r) for xi, sl in xis]
    h1 = jnp.concatenate([p[0] for p in parts], axis=0)
    s1 = sum(p[1] for p in parts)
    q1 = sum(p[2] for p in parts)
    h2, s2, q2 = _mlp2(h1, s1, q1, g1.reshape(1, HID), be1.reshape(1, HID),
                       W2, b2.reshape(1, NOUT))
    out = _bn2(h2, s2, q2, g2.reshape(1, NOUT), be2.reshape(1, NOUT))
    return (out, pos_skip, batch_skip)


# speedup vs baseline: 1.0292x; 1.0163x over previous
import functools

import jax
import jax.numpy as jnp
from jax import lax
from jax.experimental import pallas as pl
from jax.experimental.pallas import tpu as pltpu
from jax.experimental.pallas import tpu_sc as plsc

N_COARSE = 4096
N_FINE = 16384
NIN = 512
NSKIP = 256
NOUT = 512
HID = (NIN + NSKIP + NOUT) // 2
KNN = 3

BF = 1024
BM = 512

_NC = 2
_NS = 16
_NW = _NC * _NS
_PW = N_FINE // _NW
_CH = 16
_NCHUNK = _PW // _CH


def _knn_body(ps_ref, posT_ref, idx_ref, w_ref):
    ps = ps_ref[...]
    posT = posT_ref[...]
    dot = lax.dot_general(ps, posT, (((1,), (0,)), ((), ())),
                          preferred_element_type=jnp.float32)
    p2 = jnp.sum(posT * posT, axis=0, keepdims=True)
    ps2 = jnp.sum(ps * ps, axis=1, keepdims=True)
    d2 = jnp.maximum(ps2 + p2 - 2.0 * dot, 0.0)
    big = jnp.float32(3.0e38)
    nlane = 128
    nchk = N_COARSE // nlane
    av = jnp.full((d2.shape[0], nlane), big, jnp.float32)
    bv = av
    cv = av
    ai = jnp.zeros((d2.shape[0], nlane), jnp.int32)
    bi = ai
    ci = ai
    for c in range(nchk):
        xv = lax.slice(d2, (0, c * nlane), (d2.shape[0], (c + 1) * nlane))
        xc = jnp.int32(c)
        c1 = xv < av
        tv = jnp.where(c1, av, xv)
        ti = jnp.where(c1, ai, xc)
        av = jnp.minimum(av, xv)
        ai = jnp.where(c1, xc, ai)
        c2 = tv < bv
        uv = jnp.where(c2, bv, tv)
        ui = jnp.where(c2, bi, ti)
        bv = jnp.minimum(bv, tv)
        bi = jnp.where(c2, ti, bi)
        c3 = uv < cv
        cv = jnp.minimum(cv, uv)
        ci = jnp.where(c3, ui, ci)
    vals3 = jnp.concatenate([av, bv, cv], axis=1)
    lane = jnp.bitwise_and(
        lax.broadcasted_iota(jnp.int32, vals3.shape, 1), nlane - 1)
    gidx = jnp.concatenate([ai, bi, ci], axis=1) * nlane + lane
    inds, vals = [], []
    for _ in range(KNN):
        m = jnp.min(vals3, axis=1, keepdims=True)
        cand = jnp.where(vals3 == m, gidx, jnp.int32(2**30))
        g = jnp.min(cand, axis=1, keepdims=True)
        inds.append(g)
        vals.append(m)
        vals3 = jnp.where(gidx == g, big, vals3)
    ws = [1.0 / jnp.maximum(v, 1e-16) for v in vals]
    wsum = ws[0] + ws[1] + ws[2]
    idx_ref[...] = jnp.concatenate(inds, axis=1)
    w_ref[...] = jnp.concatenate([w / wsum for w in ws], axis=1)


def _knn(pos_skip, posT):
    n = pos_skip.shape[0]
    return pl.pallas_call(
        _knn_body,
        grid=(n // BF,),
        in_specs=[
            pl.BlockSpec((BF, 3), lambda i: (i, 0)),
            pl.BlockSpec((3, N_COARSE), lambda i: (0, 0)),
        ],
        out_specs=[
            pl.BlockSpec((BF, KNN), lambda i: (i, 0)),
            pl.BlockSpec((BF, KNN), lambda i: (i, 0)),
        ],
        out_shape=[
            jax.ShapeDtypeStruct((n, KNN), jnp.int32),
            jax.ShapeDtypeStruct((n, KNN), jnp.float32),
        ],
    )(pos_skip, posT)


_NBUF = 2


def _gather_body(pw, x_hbm, idx_hbm, w_hbm, xi_hbm, idxv, wv, *scr):
    nchunk = pw // _CH
    wid = lax.axis_index("s") * _NC + lax.axis_index("c")
    base = wid * pw
    pltpu.sync_copy(idx_hbm.at[pl.ds(base * KNN, KNN * pw)], idxv)
    pltpu.sync_copy(w_hbm.at[pl.ds(base * KNN, KNN * pw)],
                    wv.at[pl.ds(0, KNN * pw)])

    rows = scr[:_NBUF]
    outb = scr[_NBUF:_NBUF + 2]
    gsem = scr[_NBUF + 2:2 * _NBUF + 2]
    ssem = scr[2 * _NBUF + 2:]

    def issue_gather(ch, b):
        pltpu.async_copy(
            x_hbm.at[idxv.at[pl.ds(KNN * _CH * ch, KNN * _CH)]],
            rows[b], gsem[b])

    def gather_wait(b):
        pltpu.make_async_copy(x_hbm.at[pl.ds(0, KNN * _CH)], rows[b],
                              gsem[b]).wait()

    def store_wait(ob):
        pltpu.make_async_copy(outb[ob], xi_hbm.at[pl.ds(0, _CH)],
                              ssem[ob]).wait()

    for b in range(_NBUF):
        issue_gather(b, b)

    def quad_body(p, carry):
        for b in range(_NBUF):
            ch = _NBUF * p + b
            cb = ch * _CH
            ob = b % 2
            gather_wait(b)

            @pl.when(ch >= 2)
            def _():
                store_wait(ob)

            def point_body(i, c2):
                i3 = KNN * i
                g = KNN * cb + i3
                v = wv[pl.ds(g, 16)]
                w0 = v[0]
                w1 = v[1]
                w2 = v[2]
                rb = rows[b]
                obr = outb[ob]
                for d in range(NIN // 16):
                    sl = pl.ds(d * 16, 16)
                    obr[i, sl] = (rb[i3, sl] * w0 + rb[i3 + 1, sl] * w1
                                  + rb[i3 + 2, sl] * w2)
                return c2

            lax.fori_loop(0, _CH, point_body, 0)

            @pl.when(ch + _NBUF < nchunk)
            def _():
                issue_gather(ch + _NBUF, b)

            pltpu.async_copy(outb[ob], xi_hbm.at[pl.ds(base + cb, _CH)],
                             ssem[ob])
        return carry

    lax.fori_loop(0, nchunk // _NBUF, quad_body, 0)
    store_wait(0)
    store_wait(1)


@functools.lru_cache(maxsize=None)
def _gather_combine_fn(npoints):
    pw = npoints // _NW
    return functools.partial(
        pl.kernel,
        mesh=plsc.VectorSubcoreMesh(core_axis_name="c", subcore_axis_name="s"),
        out_type=jax.ShapeDtypeStruct((npoints, NIN), jnp.float32),
        scratch_types=[
            pltpu.VMEM((KNN * pw,), jnp.int32),
            pltpu.VMEM((KNN * pw + 16,), jnp.float32),
        ] + [pltpu.VMEM((KNN * _CH, NIN), jnp.float32)] * _NBUF + [
            pltpu.VMEM((_CH, NIN), jnp.float32),
            pltpu.VMEM((_CH, NIN), jnp.float32),
        ] + [pltpu.SemaphoreType.DMA] * (_NBUF + 2),
    )(functools.partial(_gather_body, pw))


def _mlp1_body(xi_ref, xs_ref, w1a_ref, w1b_ref, b1_ref, h1_ref, s_ref, q_ref):
    h = jnp.dot(xi_ref[...], w1a_ref[...], preferred_element_type=jnp.float32)
    h = h + jnp.dot(xs_ref[...], w1b_ref[...],
                    preferred_element_type=jnp.float32)
    h = jnp.maximum(h + b1_ref[...], 0.0)
    h1_ref[...] = h

    @pl.when(pl.program_id(0) == 0)
    def _():
        s_ref[...] = jnp.zeros_like(s_ref)
        q_ref[...] = jnp.zeros_like(q_ref)

    s_ref[...] += jnp.sum(h, axis=0, keepdims=True)
    q_ref[...] += jnp.sum(h * h, axis=0, keepdims=True)


def _mlp1(xi, x_skip, w1a, w1b, b1):
    n = xi.shape[0]
    return pl.pallas_call(
        _mlp1_body,
        grid=(n // BM,),
        in_specs=[
            pl.BlockSpec((BM, NIN), lambda i: (i, 0)),
            pl.BlockSpec((BM, NSKIP), lambda i: (i, 0)),
            pl.BlockSpec((NIN, HID), lambda i: (0, 0)),
            pl.BlockSpec((NSKIP, HID), lambda i: (0, 0)),
            pl.BlockSpec((1, HID), lambda i: (0, 0)),
        ],
        out_specs=[
            pl.BlockSpec((BM, HID), lambda i: (i, 0)),
            pl.BlockSpec((1, HID), lambda i: (0, 0)),
            pl.BlockSpec((1, HID), lambda i: (0, 0)),
        ],
        out_shape=[
            jax.ShapeDtypeStruct((n, HID), jnp.float32),
            jax.ShapeDtypeStruct((1, HID), jnp.float32),
            jax.ShapeDtypeStruct((1, HID), jnp.float32),
        ],
    )(xi, x_skip, w1a, w1b, b1)


def _mlp2_body(h1_ref, s_ref, q_ref, g1_ref, be1_ref, w2_ref, b2_ref,
               h2_ref, s2_ref, q2_ref):
    n = jnp.float32(N_FINE)
    mean = s_ref[...] / n
    var = q_ref[...] / n - mean * mean
    a = g1_ref[...] * lax.rsqrt(var + 1e-5)
    c = be1_ref[...] - a * mean
    hn = h1_ref[...] * a + c
    h = jnp.dot(hn, w2_ref[...], preferred_element_type=jnp.float32)
    h = jnp.maximum(h + b2_ref[...], 0.0)
    h2_ref[...] = h

    @pl.when(pl.program_id(0) == 0)
    def _():
        s2_ref[...] = jnp.zeros_like(s2_ref)
        q2_ref[...] = jnp.zeros_like(q2_ref)

    s2_ref[...] += jnp.sum(h, axis=0, keepdims=True)
    q2_ref[...] += jnp.sum(h * h, axis=0, keepdims=True)


def _mlp2(h1, s1, q1, g1, be1, w2, b2):
    return pl.pallas_call(
        _mlp2_body,
        grid=(N_FINE // BM,),
        in_specs=[
            pl.BlockSpec((BM, HID), lambda i: (i, 0)),
            pl.BlockSpec((1, HID), lambda i: (0, 0)),
            pl.BlockSpec((1, HID), lambda i: (0, 0)),
            pl.BlockSpec((1, HID), lambda i: (0, 0)),
            pl.BlockSpec((1, HID), lambda i: (0, 0)),
            pl.BlockSpec((HID, NOUT), lambda i: (0, 0)),
            pl.BlockSpec((1, NOUT), lambda i: (0, 0)),
        ],
        out_specs=[
            pl.BlockSpec((BM, NOUT), lambda i: (i, 0)),
            pl.BlockSpec((1, NOUT), lambda i: (0, 0)),
            pl.BlockSpec((1, NOUT), lambda i: (0, 0)),
        ],
        out_shape=[
            jax.ShapeDtypeStruct((N_FINE, NOUT), jnp.float32),
            jax.ShapeDtypeStruct((1, NOUT), jnp.float32),
            jax.ShapeDtypeStruct((1, NOUT), jnp.float32),
        ],
    )(h1, s1, q1, g1, be1, w2, b2)


def _bn2_body(h2_ref, s_ref, q_ref, g2_ref, be2_ref, o_ref):
    n = jnp.float32(N_FINE)
    mean = s_ref[...] / n
    var = q_ref[...] / n - mean * mean
    a = g2_ref[...] * lax.rsqrt(var + 1e-5)
    c = be2_ref[...] - a * mean
    o_ref[...] = h2_ref[...] * a + c


def _bn2(h2, s2, q2, g2, be2):
    return pl.pallas_call(
        _bn2_body,
        grid=(N_FINE // BM,),
        in_specs=[
            pl.BlockSpec((BM, NOUT), lambda i: (i, 0)),
            pl.BlockSpec((1, NOUT), lambda i: (0, 0)),
            pl.BlockSpec((1, NOUT), lambda i: (0, 0)),
            pl.BlockSpec((1, NOUT), lambda i: (0, 0)),
            pl.BlockSpec((1, NOUT), lambda i: (0, 0)),
        ],
        out_specs=pl.BlockSpec((BM, NOUT), lambda i: (i, 0)),
        out_shape=jax.ShapeDtypeStruct((N_FINE, NOUT), jnp.float32),
    )(h2, s2, q2, g2, be2)


def kernel(x, pos, batch, x_skip, pos_skip, batch_skip,
           W1, b1, g1, be1, W2, b2, g2, be2):
    posT = pos.T
    bounds = [0, N_FINE // 2, N_FINE]
    w1a, w1b = W1[:NIN], W1[NIN:]
    b1r = b1.reshape(1, HID)
    xis = []
    for lo, hi in zip(bounds[:-1], bounds[1:]):
        sl = slice(lo, hi)
        idx, w = _knn(pos_skip[sl], posT)
        xis.append((_gather_combine_fn(hi - lo)(x, idx.reshape(-1),
                                                w.reshape(-1)), sl))
    parts = [_mlp1(xi, x_skip[sl], w1a, w1b, b1r) for xi, sl in xis]
    h1 = jnp.concatenate([p[0] for p in parts], axis=0)
    s1 = sum(p[1] for p in parts)
    q1 = sum(p[2] for p in parts)
    h2, s2, q2 = _mlp2(h1, s1, q1, g1.reshape(1, HID), be1.reshape(1, HID),
                       W2, b2.reshape(1, NOUT))
    out = _bn2(h2, s2, q2, g2.reshape(1, NOUT), be2.reshape(1, NOUT))
    return (out, pos_skip, batch_skip)
